# baseline (speedup 1.0000x reference)
import functools

import jax
import jax.numpy as jnp
import numpy as np
from jax.experimental import pallas as pl
from jax.experimental.pallas import tpu as pltpu
from jax.experimental.pallas import tpu_sc as plsc

_NODE_DIM = 256
_EMB_DIM = 512
_SEQ_LEN = 2048


def _basesin(t, omiga, fai=0.0):
    T = 2 * np.pi / omiga
    return np.sin(omiga * np.abs(np.mod(t, 2 * T) - T) + fai)


def _basecos(t, omiga, fai=0.0):
    T = 2 * np.pi / omiga
    return np.cos(omiga * np.abs(np.mod(t, 2 * T) - T) + fai)


def _make_cpe(n_position, emb_dim, mean_pooling=True):
    skip_base = np.power(n_position, 1 / (emb_dim // 2))
    skip_set = np.linspace(skip_base, n_position, emb_dim // 2, dtype='int')
    x = np.zeros((n_position, emb_dim))
    for i in range(emb_dim):
        skip = skip_set[i // 3 * 3 + 1] if i // 3 * 3 + 1 < emb_dim // 2 else skip_set[-1]
        if n_position > skip:
            longer_pattern = np.arange(0, np.ceil(n_position / skip) * skip + 0.01, 0.01)
        else:
            longer_pattern = np.arange(0, n_position + 0.01, 0.01)
            skip = n_position
        num = len(longer_pattern) - 1
        omiga = 2 * np.pi / skip
        fai = 0 if i <= emb_dim // 2 else 2 * np.pi * ((-i + emb_dim // 2) / (emb_dim // 2))
        sel = np.linspace(0, num, n_position + 1, dtype='int')
        if i % 2 == 1:
            x[:, i] = _basecos(longer_pattern, omiga, fai)[sel][:n_position]
        else:
            x[:, i] = _basesin(longer_pattern, omiga, fai)[sel][:n_position]
    pattern = x.astype(np.float32)
    pattern_sum = np.zeros_like(pattern)
    arange = np.arange(n_position)
    pooling = [-2, -1, 0, 1, 2] if mean_pooling else [0]
    time = 0
    for i in pooling:
        time += 1
        index = (arange + i + n_position) % n_position
        pattern_sum += pattern[index]
    pattern = 1.0 / time * pattern_sum - pattern.mean(0)
    return pattern


_PATTERN_NP = _make_cpe(_SEQ_LEN, _EMB_DIM, mean_pooling=True)

_NC = 2
_NS = 16
_CH = 32
_NBUF = 4


def _pfe_gather(table, idx, B, S):
    mesh = plsc.VectorSubcoreMesh(core_axis_name="c", subcore_axis_name="s")
    nch = S // _CH

    @functools.partial(
        pl.kernel,
        out_type=jax.ShapeDtypeStruct((B, S, _EMB_DIM), jnp.float32),
        mesh=mesh,
        scratch_types=[
            pltpu.VMEM((S,), jnp.int32),
            pltpu.VMEM((_NBUF, _CH, _EMB_DIM), jnp.float32),
        ] + [pltpu.SemaphoreType.DMA] * (2 * _NBUF),
    )
    def k(table_hbm, i_hbm, o_hbm, idx_v, rows_v, *sems):
        gsems, osems = sems[:_NBUF], sems[_NBUF:]
        wid = jax.lax.axis_index("s") * _NC + jax.lax.axis_index("c")
        pltpu.sync_copy(i_hbm.at[wid], idx_v)
        my_out = o_hbm.at[wid]

        def gather(c, buf):
            pltpu.async_copy(
                table_hbm.at[idx_v.at[pl.ds(c * _CH, _CH)]], rows_v.at[buf],
                gsems[buf])

        def wait_gather(buf):
            pltpu.make_async_copy(
                table_hbm.at[idx_v.at[pl.ds(0, _CH)]], rows_v.at[buf],
                gsems[buf]).wait()

        def put(c, buf):
            pltpu.async_copy(rows_v.at[buf], my_out.at[pl.ds(c * _CH, _CH)],
                             osems[buf])

        def wait_put(buf):
            pltpu.make_async_copy(
                rows_v.at[buf], my_out.at[pl.ds(0, _CH)], osems[buf]).wait()

        gather(0, 0)
        gather(1, 1)

        @pl.loop(0, nch, step=_NBUF)
        def _(c):
            for j in range(_NBUF):
                bj = j % _NBUF
                bn = (j + 2) % _NBUF
                wait_gather(bj)
                put(c + j, bj)

                @pl.when((c + j + 2 < nch) & (c + j - 2 >= 0))
                def _():
                    wait_put(bn)

                @pl.when(c + j + 2 < nch)
                def _():
                    gather(c + j + 2, bn)

        for b in range(_NBUF):
            wait_put(b)

    return k(table, idx)


_MM_BM = 1024


def _nfe_matmul(x, W):
    B, S, N = x.shape

    def mm(x_ref, w_ref, o_ref):
        o_ref[0] = jax.lax.dot_general(
            x_ref[0], w_ref[...], (((1,), (1,)), ((), ())),
            preferred_element_type=jnp.float32)

    return pl.pallas_call(
        mm,
        grid=(B, S // _MM_BM),
        in_specs=[
            pl.BlockSpec((1, _MM_BM, N), lambda b, j: (b, j, 0)),
            pl.BlockSpec((_EMB_DIM, N), lambda b, j: (0, 0)),
        ],
        out_specs=pl.BlockSpec((1, _MM_BM, _EMB_DIM), lambda b, j: (b, j, 0)),
        out_shape=jax.ShapeDtypeStruct((B, S, _EMB_DIM), jnp.float32),
    )(x, W)


def kernel(x, solutions, visited_time, W):
    B, S = visited_time.shape
    table = jnp.asarray(_PATTERN_NP)
    idx = jnp.mod(visited_time, S).astype(jnp.int32)
    PFEs = _pfe_gather(table, idx, B, S)
    NFEs = _nfe_matmul(x, W)
    return (NFEs, PFEs, visited_time.astype(jnp.int64))


# speedup vs baseline: 1.0125x; 1.0125x over previous
import functools

import jax
import jax.numpy as jnp
import numpy as np
from jax.experimental import pallas as pl
from jax.experimental.pallas import tpu as pltpu
from jax.experimental.pallas import tpu_sc as plsc

_NODE_DIM = 256
_EMB_DIM = 512
_SEQ_LEN = 2048


def _basesin(t, omiga, fai=0.0):
    T = 2 * np.pi / omiga
    return np.sin(omiga * np.abs(np.mod(t, 2 * T) - T) + fai)


def _basecos(t, omiga, fai=0.0):
    T = 2 * np.pi / omiga
    return np.cos(omiga * np.abs(np.mod(t, 2 * T) - T) + fai)


def _make_cpe(n_position, emb_dim, mean_pooling=True):
    skip_base = np.power(n_position, 1 / (emb_dim // 2))
    skip_set = np.linspace(skip_base, n_position, emb_dim // 2, dtype='int')
    x = np.zeros((n_position, emb_dim))
    for i in range(emb_dim):
        skip = skip_set[i // 3 * 3 + 1] if i // 3 * 3 + 1 < emb_dim // 2 else skip_set[-1]
        if n_position > skip:
            longer_pattern = np.arange(0, np.ceil(n_position / skip) * skip + 0.01, 0.01)
        else:
            longer_pattern = np.arange(0, n_position + 0.01, 0.01)
            skip = n_position
        num = len(longer_pattern) - 1
        omiga = 2 * np.pi / skip
        fai = 0 if i <= emb_dim // 2 else 2 * np.pi * ((-i + emb_dim // 2) / (emb_dim // 2))
        sel = np.linspace(0, num, n_position + 1, dtype='int')
        if i % 2 == 1:
            x[:, i] = _basecos(longer_pattern, omiga, fai)[sel][:n_position]
        else:
            x[:, i] = _basesin(longer_pattern, omiga, fai)[sel][:n_position]
    pattern = x.astype(np.float32)
    pattern_sum = np.zeros_like(pattern)
    arange = np.arange(n_position)
    pooling = [-2, -1, 0, 1, 2] if mean_pooling else [0]
    time = 0
    for i in pooling:
        time += 1
        index = (arange + i + n_position) % n_position
        pattern_sum += pattern[index]
    pattern = 1.0 / time * pattern_sum - pattern.mean(0)
    return pattern


_PATTERN_NP = _make_cpe(_SEQ_LEN, _EMB_DIM, mean_pooling=True)

_NC = 2
_NS = 16
_CH = 32
_NBUF = 4


def _pfe_gather(table, idx, B, S):
    mesh = plsc.VectorSubcoreMesh(core_axis_name="c", subcore_axis_name="s")
    nch = S // _CH

    @functools.partial(
        pl.kernel,
        out_type=jax.ShapeDtypeStruct((B, S, _EMB_DIM), jnp.float32),
        mesh=mesh,
        scratch_types=[
            pltpu.VMEM((S,), jnp.int32),
            pltpu.VMEM((_NBUF, _CH, _EMB_DIM), jnp.float32),
        ] + [pltpu.SemaphoreType.DMA] * (2 * _NBUF),
    )
    def k(table_hbm, i_hbm, o_hbm, idx_v, rows_v, *sems):
        gsems, osems = sems[:_NBUF], sems[_NBUF:]
        wid = jax.lax.axis_index("s") * _NC + jax.lax.axis_index("c")
        pltpu.sync_copy(i_hbm.at[wid], idx_v)
        my_out = o_hbm.at[wid]

        def mask(c):
            for t in range(_CH // 16):
                sl = pl.ds(c * _CH + t * 16, 16)
                idx_v[sl] = jax.lax.bitwise_and(idx_v[sl], S - 1)

        def gather(c, buf):
            pltpu.async_copy(
                table_hbm.at[idx_v.at[pl.ds(c * _CH, _CH)]], rows_v.at[buf],
                gsems[buf])

        def wait_gather(buf):
            pltpu.make_async_copy(
                table_hbm.at[idx_v.at[pl.ds(0, _CH)]], rows_v.at[buf],
                gsems[buf]).wait()

        def put(c, buf):
            pltpu.async_copy(rows_v.at[buf], my_out.at[pl.ds(c * _CH, _CH)],
                             osems[buf])

        def wait_put(buf):
            pltpu.make_async_copy(
                rows_v.at[buf], my_out.at[pl.ds(0, _CH)], osems[buf]).wait()

        mask(0)
        gather(0, 0)
        mask(1)
        gather(1, 1)

        @pl.loop(0, nch, step=_NBUF)
        def _(c):
            for j in range(_NBUF):
                bj = j % _NBUF
                bn = (j + 2) % _NBUF
                wait_gather(bj)
                put(c + j, bj)

                @pl.when((c + j + 2 < nch) & (c + j - 2 >= 0))
                def _():
                    wait_put(bn)

                @pl.when(c + j + 2 < nch)
                def _():
                    mask(c + j + 2)
                    gather(c + j + 2, bn)

        for b in range(_NBUF):
            wait_put(b)

    return k(table, idx)


_MM_BM = 1024


def _nfe_matmul(x, W):
    B, S, N = x.shape

    def mm(x_ref, w_ref, o_ref):
        o_ref[0] = jax.lax.dot_general(
            x_ref[0], w_ref[...], (((1,), (1,)), ((), ())),
            preferred_element_type=jnp.float32)

    return pl.pallas_call(
        mm,
        grid=(B, S // _MM_BM),
        in_specs=[
            pl.BlockSpec((1, _MM_BM, N), lambda b, j: (b, j, 0)),
            pl.BlockSpec((_EMB_DIM, N), lambda b, j: (0, 0)),
        ],
        out_specs=pl.BlockSpec((1, _MM_BM, _EMB_DIM), lambda b, j: (b, j, 0)),
        out_shape=jax.ShapeDtypeStruct((B, S, _EMB_DIM), jnp.float32),
    )(x, W)


def kernel(x, solutions, visited_time, W):
    B, S = visited_time.shape
    table = jnp.asarray(_PATTERN_NP)
    idx = visited_time.astype(jnp.int32)
    PFEs = _pfe_gather(table, idx, B, S)
    NFEs = _nfe_matmul(x, W)
    return (NFEs, PFEs, visited_time.astype(jnp.int64))
